# Initial kernel scaffold; baseline (speedup 1.0000x reference)
import jax
import jax.numpy as jnp
from jax.experimental import pallas as pl


def kernel(f, node_features, edge_index, edge_weight, w_fw1, b_fw1, w_fw2, b_fw2, w_se1, b_se1, w_se2, b_se2, w_cn1, b_cn1, w_cn2, b_cn2, w_cn3, b_cn3):
    raise NotImplementedError("write your pallas kernel here")


# speedup vs baseline: 5.6415x; 5.6415x over previous
import functools

import jax
import jax.numpy as jnp
from jax import lax
from jax.experimental import pallas as pl
from jax.experimental.pallas import tpu as pltpu
from jax.experimental.pallas import tpu_sc as plsc

_N = 10000
_E = 320000
_Q = 20
_FD = 128
_H = 64
_MAX_V = 20.0
_DT = 0.1

_NC = 2
_NS = 16
_NW = _NC * _NS
_CH = 128
_NCHUNK = _E // _CH
_CBASE = _NCHUNK // _NW
_CREM = _NCHUNK - _CBASE * _NW
_RPT = _N // _NS


def _tc_proj_body(x_ref, wsT_ref, wdT_ref, b1_ref, ps_ref, pd_ref):
    x = x_ref[...]
    ps_ref[...] = (
        jnp.dot(x, wsT_ref[...], precision=jax.lax.Precision.HIGHEST)
        + b1_ref[...]
    )
    pd_ref[...] = jnp.dot(x, wdT_ref[...], precision=jax.lax.Precision.HIGHEST)


def _tc_proj(nf, wsT, wdT, b1):
    blk = 1000
    grid = _N // blk
    return pl.pallas_call(
        _tc_proj_body,
        grid=(grid,),
        in_specs=[
            pl.BlockSpec((blk, _FD), lambda i: (i, 0)),
            pl.BlockSpec((_FD, _FD), lambda i: (0, 0)),
            pl.BlockSpec((_FD, _FD), lambda i: (0, 0)),
            pl.BlockSpec((1, _FD), lambda i: (0, 0)),
        ],
        out_specs=[
            pl.BlockSpec((blk, _FD), lambda i: (i, 0)),
            pl.BlockSpec((blk, _FD), lambda i: (i, 0)),
        ],
        out_shape=[
            jax.ShapeDtypeStruct((_N, _FD), jnp.float32),
            jax.ShapeDtypeStruct((_N, _FD), jnp.float32),
        ],
    )(nf, wsT, wdT, b1)


def _tc_mid_body(f_ref, ws_ref, raw_ref, ew_ref, s_ref, g_ref):
    ws = ws_ref[0, :, 0:1] + ws_ref[1, :, 0:1]
    inv = 1.0 / jnp.maximum(ws, 1e-8)
    f = f_ref[...]
    s_ref[...] = jnp.concatenate(
        [f, jnp.zeros((f.shape[0], 32 - _Q - 1), jnp.float32), inv], axis=-1
    )
    g_ref[...] = raw_ref[...] / jnp.maximum(ew_ref[...], 1e-8)


def _tc_mid(f, ws_part, raw2d, ew2d):
    return pl.pallas_call(
        _tc_mid_body,
        out_shape=[
            jax.ShapeDtypeStruct((_N, 32), jnp.float32),
            jax.ShapeDtypeStruct(raw2d.shape, jnp.float32),
        ],
    )(f, ws_part, raw2d, ew2d)


def _tc_final_body(acc_ref, f_ref, wse1_ref, bse1_ref, wse2_ref, bse2_ref,
                   wcn1_ref, bcn1_ref, wcn2_ref, bcn2_ref, wcn3_ref, bcn3_ref,
                   out_ref):
    a = acc_ref[0] + acc_ref[1]
    transport = a[:, 0:_Q]
    neigh = a[:, 32:32 + _Q]
    deg = a[:, 52:53]
    f = f_ref[...]
    hp = jax.lax.Precision.HIGHEST
    nf_avg = neigh / jnp.maximum(deg, 1.0)
    sp = jnp.maximum(jnp.dot(nf_avg, wse1_ref[...], precision=hp)
                     + bse1_ref[...], 0.0)
    sp = jnp.dot(sp, wse2_ref[...], precision=hp) + bse2_ref[...]
    ci = jnp.concatenate([f, sp], axis=-1)
    o = jnp.maximum(jnp.dot(ci, wcn1_ref[...], precision=hp)
                    + bcn1_ref[...], 0.0)
    o = jnp.maximum(jnp.dot(o, wcn2_ref[...], precision=hp)
                    + bcn2_ref[...], 0.0)
    omega = jnp.dot(o, wcn3_ref[...], precision=hp) + bcn3_ref[...]
    out_ref[...] = f + _DT * (omega - transport)


def _tc_final(acc, f, wse1T, bse1, wse2T, bse2, wcn1T, bcn1, wcn2T, bcn2,
              wcn3T, bcn3):
    blk = 1000
    grid = _N // blk
    wspec = lambda shape: pl.BlockSpec(shape, lambda i: tuple(0 for _ in shape))
    return pl.pallas_call(
        _tc_final_body,
        grid=(grid,),
        in_specs=[
            pl.BlockSpec((2, blk, 64), lambda i: (0, i, 0)),
            pl.BlockSpec((blk, _Q), lambda i: (i, 0)),
            wspec(wse1T.shape), wspec(bse1.shape),
            wspec(wse2T.shape), wspec(bse2.shape),
            wspec(wcn1T.shape), wspec(bcn1.shape),
            wspec(wcn2T.shape), wspec(bcn2.shape),
            wspec(wcn3T.shape), wspec(bcn3.shape),
        ],
        out_specs=pl.BlockSpec((blk, _Q), lambda i: (i, 0)),
        out_shape=jax.ShapeDtypeStruct((_N, _Q), jnp.float32),
    )(acc, f, wse1T, bse1, wse2T, bse2, wcn1T, bcn1, wcn2T, bcn2, wcn3T, bcn3)


def _sc_a_body(sidx_hbm, didx_hbm, ew_hbm, ps_hbm, pd_hbm, wl_hbm, w2_hbm,
               b2_hbm,
               raw_hbm, ws_out,
               sidx_v, didx_v, ew_v, ps_buf, pd_buf, rawlin_v, rawrow_v,
               wl_v, w2_v, b2_v, zbuf, ws_sh, sem1, sem2):
    cid = lax.axis_index("c")
    sid = lax.axis_index("s")
    wid = sid * _NC + cid

    zv = jnp.zeros((16,), jnp.float32)

    def zrow(j, _):
        zbuf[j, :] = zv
        return ()
    lax.fori_loop(0, _RPT, zrow, ())

    def zrow2(j, _):
        rawrow_v[j, :] = zv
        return ()
    lax.fori_loop(0, _CH, zrow2, ())

    pltpu.sync_copy(zbuf, ws_sh.at[pl.ds(sid * _RPT, _RPT)])
    plsc.subcore_barrier()

    pltpu.sync_copy(wl_hbm, wl_v)
    pltpu.sync_copy(w2_hbm, w2_v)
    pltpu.sync_copy(b2_hbm, b2_v)
    wlk = [wl_v[pl.ds(16 * k, 16)] for k in range(8)]
    w2k = [w2_v[pl.ds(16 * k, 16)] for k in range(8)]
    b2s = b2_v[...][0]
    lanes = jnp.arange(16, dtype=jnp.int32)
    zeros_i = jnp.zeros((16,), jnp.int32)
    m0 = lanes == 0

    nch = jnp.where(wid < _CREM, _CBASE + 1, _CBASE)

    def chunk_body(k, _):
        chunk = wid + _NW * k
        ebase = chunk * _CH
        pltpu.sync_copy(sidx_hbm.at[pl.ds(ebase, _CH)], sidx_v)
        pltpu.sync_copy(didx_hbm.at[pl.ds(ebase, _CH)], didx_v)
        pltpu.sync_copy(ew_hbm.at[pl.ds(ebase, _CH)], ew_v.at[pl.ds(0, _CH)])
        cp1 = pltpu.async_copy(ps_hbm.at[sidx_v], ps_buf, sem1)
        cp2 = pltpu.async_copy(pd_hbm.at[didx_v], pd_buf, sem2)
        cp1.wait()
        cp2.wait()

        def edge_body(e, _):
            ew_e = ew_v[pl.ds(e, 16)][0]
            acc = jnp.zeros((16,), jnp.float32)
            for kk in range(8):
                a = ps_buf[e, pl.ds(16 * kk, 16)]
                b = pd_buf[e, pl.ds(16 * kk, 16)]
                t = jnp.maximum(a + b + ew_e * wlk[kk], 0.0)
                acc = acc + t * w2k[kk]
            x = jnp.sum(acc) + b2s
            plsc.store_scatter(
                rawlin_v, [jnp.broadcast_to(e, (16,)).astype(jnp.int32)],
                jnp.broadcast_to(x, (16,)), mask=m0)
            return ()
        lax.fori_loop(0, _CH, edge_body, ())

        for i in range(_CH // 16):
            xv = rawlin_v[pl.ds(16 * i, 16)]
            rv = 1.0 / (1.0 + jnp.exp(-xv))
            rawlin_v[pl.ds(16 * i, 16)] = rv
            plsc.store_scatter(rawrow_v, [lanes + 16 * i, zeros_i], rv)

        pltpu.sync_copy(rawlin_v, raw_hbm.at[pl.ds(ebase, _CH)])
        pltpu.sync_copy(rawrow_v, ws_sh.at[didx_v], add=True)
        return ()

    lax.fori_loop(0, nch, chunk_body, ())

    plsc.subcore_barrier()

    @pl.when(sid == 0)
    def _():
        pltpu.sync_copy(ws_sh, ws_out.at[cid])


def _sc_a(sidx, didx, ew, ps, pd, wl, w2, b2):
    mesh = plsc.VectorSubcoreMesh(core_axis_name="c", subcore_axis_name="s")
    fn = pl.kernel(
        _sc_a_body,
        out_type=[
            jax.ShapeDtypeStruct((_E,), jnp.float32),
            jax.ShapeDtypeStruct((_NC, _N, 16), jnp.float32),
        ],
        mesh=mesh,
        scratch_types=[
            pltpu.VMEM((_CH,), jnp.int32),
            pltpu.VMEM((_CH,), jnp.int32),
            pltpu.VMEM((_CH + 16,), jnp.float32),
            pltpu.VMEM((_CH, _FD), jnp.float32),
            pltpu.VMEM((_CH, _FD), jnp.float32),
            pltpu.VMEM((_CH,), jnp.float32),
            pltpu.VMEM((_CH, 16), jnp.float32),
            pltpu.VMEM((_FD,), jnp.float32),
            pltpu.VMEM((_FD,), jnp.float32),
            pltpu.VMEM((16,), jnp.float32),
            pltpu.VMEM((_RPT, 16), jnp.float32),
            pltpu.VMEM_SHARED((_N, 16), jnp.float32),
            pltpu.SemaphoreType.DMA,
            pltpu.SemaphoreType.DMA,
        ],
        compiler_params=pltpu.CompilerParams(needs_layout_passes=False, use_tc_tiling_on_sc=False),
    )
    return fn(sidx, didx, ew, ps, pd, wl, w2, b2)


def _sc_b_body(sidx_hbm, didx_hbm, g_hbm, s_hbm, v_hbm,
               acc_out,
               sidx_v, didx_v, g_v, sbuf, dbuf, srcrow_v, v_v, zbuf, acc_sh,
               sem1, sem2):
    cid = lax.axis_index("c")
    sid = lax.axis_index("s")
    wid = sid * _NC + cid

    zv = jnp.zeros((16,), jnp.float32)

    def zrow(j, _):
        for kk in range(4):
            zbuf[j, pl.ds(16 * kk, 16)] = zv
        return ()
    lax.fori_loop(0, _RPT, zrow, ())
    pltpu.sync_copy(zbuf, acc_sh.at[pl.ds(sid * _RPT, _RPT)])
    plsc.subcore_barrier()

    pltpu.sync_copy(v_hbm, v_v)
    vlo = v_v[pl.ds(0, 16)]
    vhi = v_v[pl.ds(16, 16)]
    lanes = jnp.arange(16, dtype=jnp.int32)
    degv = jnp.where(lanes == _Q - 16, 1.0, 0.0).astype(jnp.float32)

    nch = jnp.where(wid < _CREM, _CBASE + 1, _CBASE)

    def chunk_body(k, _):
        chunk = wid + _NW * k
        ebase = chunk * _CH
        pltpu.sync_copy(sidx_hbm.at[pl.ds(ebase, _CH)], sidx_v)
        pltpu.sync_copy(didx_hbm.at[pl.ds(ebase, _CH)], didx_v)
        pltpu.sync_copy(g_hbm.at[pl.ds(ebase, _CH)], g_v.at[pl.ds(0, _CH)])
        cp1 = pltpu.async_copy(s_hbm.at[sidx_v], sbuf, sem1)
        cp2 = pltpu.async_copy(s_hbm.at[didx_v], dbuf, sem2)
        cp1.wait()
        cp2.wait()

        def edge_body(e, _):
            fs_lo = sbuf[e, pl.ds(0, 16)]
            fs_hi = sbuf[e, pl.ds(16, 16)]
            c = g_v[pl.ds(e, 16)][0] * fs_hi[15]
            fd_lo = dbuf[e, pl.ds(0, 16)]
            fd_hi = dbuf[e, pl.ds(16, 16)]
            srcrow_v[e, pl.ds(0, 16)] = (fd_lo - fs_lo) * vlo * c
            srcrow_v[e, pl.ds(16, 16)] = (fd_hi - fs_hi) * vhi * c
            srcrow_v[e, pl.ds(32, 16)] = fs_lo
            srcrow_v[e, pl.ds(48, 16)] = fs_hi + degv
            return ()
        lax.fori_loop(0, _CH, edge_body, ())

        pltpu.sync_copy(srcrow_v, acc_sh.at[didx_v], add=True)
        return ()

    lax.fori_loop(0, nch, chunk_body, ())

    plsc.subcore_barrier()

    @pl.when(sid == 0)
    def _():
        pltpu.sync_copy(acc_sh, acc_out.at[cid])


def _sc_b(sidx, didx, g, s_tab, vpad):
    mesh = plsc.VectorSubcoreMesh(core_axis_name="c", subcore_axis_name="s")
    fn = pl.kernel(
        _sc_b_body,
        out_type=jax.ShapeDtypeStruct((_NC, _N, 64), jnp.float32),
        mesh=mesh,
        scratch_types=[
            pltpu.VMEM((_CH,), jnp.int32),
            pltpu.VMEM((_CH,), jnp.int32),
            pltpu.VMEM((_CH + 16,), jnp.float32),
            pltpu.VMEM((_CH, 32), jnp.float32),
            pltpu.VMEM((_CH, 32), jnp.float32),
            pltpu.VMEM((_CH, 64), jnp.float32),
            pltpu.VMEM((32,), jnp.float32),
            pltpu.VMEM((_RPT, 64), jnp.float32),
            pltpu.VMEM_SHARED((_N, 64), jnp.float32),
            pltpu.SemaphoreType.DMA,
            pltpu.SemaphoreType.DMA,
        ],
        compiler_params=pltpu.CompilerParams(needs_layout_passes=False, use_tc_tiling_on_sc=False),
    )
    return fn(sidx, didx, g, s_tab, vpad)


@jax.jit
def kernel(f, node_features, edge_index, edge_weight, w_fw1, b_fw1, w_fw2,
           b_fw2, w_se1, b_se1, w_se2, b_se2, w_cn1, b_cn1, w_cn2, b_cn2,
           w_cn3, b_cn3):
    sidx = edge_index[0]
    didx = edge_index[1]

    wsT = w_fw1[:, :_FD].T
    wdT = w_fw1[:, _FD:2 * _FD].T
    wl = w_fw1[:, 2 * _FD]
    w2 = w_fw2[0]
    b2 = jnp.broadcast_to(b_fw2, (16,))

    ps, pd = _tc_proj(node_features, wsT, wdT, b_fw1.reshape(1, _FD))

    raw, ws_part = _sc_a(sidx, didx, edge_weight, ps, pd, wl, w2, b2)

    raw2d = raw.reshape(_E // _FD, _FD)
    ew2d = edge_weight.reshape(_E // _FD, _FD)
    s_tab, g2d = _tc_mid(f, ws_part, raw2d, ew2d)
    g = g2d.reshape(_E)

    vpad = jnp.concatenate(
        [jnp.linspace(0.0, _MAX_V, _Q, dtype=jnp.float32),
         jnp.zeros((12,), jnp.float32)]
    )

    acc = _sc_b(sidx, didx, g, s_tab, vpad)

    f_new = _tc_final(
        acc, f,
        w_se1.T, b_se1.reshape(1, _H), w_se2.T, b_se2.reshape(1, _H),
        w_cn1.T, b_cn1.reshape(1, _H), w_cn2.T, b_cn2.reshape(1, _H),
        w_cn3.T, b_cn3.reshape(1, _Q),
    )
    return f_new


# speedup vs baseline: 8.9582x; 1.5879x over previous
import functools

import jax
import jax.numpy as jnp
from jax import lax
from jax.experimental import pallas as pl
from jax.experimental.pallas import tpu as pltpu
from jax.experimental.pallas import tpu_sc as plsc

_N = 10000
_E = 320000
_Q = 20
_FD = 128
_H = 64
_MAX_V = 20.0
_DT = 0.1

_NC = 2
_NS = 16
_NW = _NC * _NS
_CH = 128
_NCHUNK = _E // _CH
_CBASE = _NCHUNK // _NW
_CREM = _NCHUNK - _CBASE * _NW
_RPT = _N // _NS


def _tc_proj_body(x_ref, wsT_ref, wdT_ref, b1_ref, ps_ref, pd_ref):
    x = x_ref[...]
    ps_ref[...] = (
        jnp.dot(x, wsT_ref[...], precision=jax.lax.Precision.HIGHEST)
        + b1_ref[...]
    )
    pd_ref[...] = jnp.dot(x, wdT_ref[...], precision=jax.lax.Precision.HIGHEST)


def _tc_proj(nf, wsT, wdT, b1):
    blk = 1000
    grid = _N // blk
    return pl.pallas_call(
        _tc_proj_body,
        grid=(grid,),
        in_specs=[
            pl.BlockSpec((blk, _FD), lambda i: (i, 0)),
            pl.BlockSpec((_FD, _FD), lambda i: (0, 0)),
            pl.BlockSpec((_FD, _FD), lambda i: (0, 0)),
            pl.BlockSpec((1, _FD), lambda i: (0, 0)),
        ],
        out_specs=[
            pl.BlockSpec((blk, _FD), lambda i: (i, 0)),
            pl.BlockSpec((blk, _FD), lambda i: (i, 0)),
        ],
        out_shape=[
            jax.ShapeDtypeStruct((_N, _FD), jnp.float32),
            jax.ShapeDtypeStruct((_N, _FD), jnp.float32),
        ],
    )(nf, wsT, wdT, b1)


def _tc_mid_body(f_ref, ws_ref, raw_ref, ew_ref, s_ref, g_ref):
    ws = ws_ref[0, :, 0:1] + ws_ref[1, :, 0:1]
    inv = 1.0 / jnp.maximum(ws, 1e-8)
    f = f_ref[...]
    s_ref[...] = jnp.concatenate(
        [f, jnp.zeros((f.shape[0], 32 - _Q - 1), jnp.float32), inv], axis=-1
    )
    g_ref[...] = raw_ref[...] / jnp.maximum(ew_ref[...], 1e-8)


def _tc_mid(f, ws_part, raw2d, ew2d):
    return pl.pallas_call(
        _tc_mid_body,
        out_shape=[
            jax.ShapeDtypeStruct((_N, 32), jnp.float32),
            jax.ShapeDtypeStruct(raw2d.shape, jnp.float32),
        ],
    )(f, ws_part, raw2d, ew2d)


def _tc_final_body(acc_ref, f_ref, wse1_ref, bse1_ref, wse2_ref, bse2_ref,
                   wcn1_ref, bcn1_ref, wcn2_ref, bcn2_ref, wcn3_ref, bcn3_ref,
                   out_ref):
    a = acc_ref[0] + acc_ref[1]
    transport = a[:, 0:_Q]
    neigh = a[:, 32:32 + _Q]
    deg = a[:, 52:53]
    f = f_ref[...]
    hp = jax.lax.Precision.HIGHEST
    nf_avg = neigh / jnp.maximum(deg, 1.0)
    sp = jnp.maximum(jnp.dot(nf_avg, wse1_ref[...], precision=hp)
                     + bse1_ref[...], 0.0)
    sp = jnp.dot(sp, wse2_ref[...], precision=hp) + bse2_ref[...]
    ci = jnp.concatenate([f, sp], axis=-1)
    o = jnp.maximum(jnp.dot(ci, wcn1_ref[...], precision=hp)
                    + bcn1_ref[...], 0.0)
    o = jnp.maximum(jnp.dot(o, wcn2_ref[...], precision=hp)
                    + bcn2_ref[...], 0.0)
    omega = jnp.dot(o, wcn3_ref[...], precision=hp) + bcn3_ref[...]
    out_ref[...] = f + _DT * (omega - transport)


def _tc_final(acc, f, wse1T, bse1, wse2T, bse2, wcn1T, bcn1, wcn2T, bcn2,
              wcn3T, bcn3):
    blk = 1000
    grid = _N // blk
    wspec = lambda shape: pl.BlockSpec(shape, lambda i: tuple(0 for _ in shape))
    return pl.pallas_call(
        _tc_final_body,
        grid=(grid,),
        in_specs=[
            pl.BlockSpec((2, blk, 64), lambda i: (0, i, 0)),
            pl.BlockSpec((blk, _Q), lambda i: (i, 0)),
            wspec(wse1T.shape), wspec(bse1.shape),
            wspec(wse2T.shape), wspec(bse2.shape),
            wspec(wcn1T.shape), wspec(bcn1.shape),
            wspec(wcn2T.shape), wspec(bcn2.shape),
            wspec(wcn3T.shape), wspec(bcn3.shape),
        ],
        out_specs=pl.BlockSpec((blk, _Q), lambda i: (i, 0)),
        out_shape=jax.ShapeDtypeStruct((_N, _Q), jnp.float32),
    )(acc, f, wse1T, bse1, wse2T, bse2, wcn1T, bcn1, wcn2T, bcn2, wcn3T, bcn3)


def _sc_a_body(edata_hbm, ps_hbm, pd_hbm, wl_hbm, w2_hbm, b2_hbm,
               raw_hbm, ws_out,
               ebuf0, ebuf1, ps0, ps1, pd0, pd1, didx0, didx1, ewf,
               xacc, rawlin0, rawlin1, rawrow0, rawrow1,
               wl_v, w2_v, b2_v, zbuf, ws_sh,
               sgp0, sgp1, sgd0, sgd1, sst0, sst1, ssc0, ssc1):
    cid = lax.axis_index("c")
    sid = lax.axis_index("s")
    wid = sid * _NC + cid

    ebuf = [ebuf0, ebuf1]
    psb = [ps0, ps1]
    pdb = [pd0, pd1]
    didxb = [didx0, didx1]
    rawlinb = [rawlin0, rawlin1]
    rawrowb = [rawrow0, rawrow1]
    sgp = [sgp0, sgp1]
    sgd = [sgd0, sgd1]
    sst = [sst0, sst1]
    ssc = [ssc0, ssc1]

    zv = jnp.zeros((16,), jnp.float32)

    def zrow(j, _):
        zbuf[j, :] = zv
        return ()
    lax.fori_loop(0, _RPT, zrow, ())

    def zrow2(j, _):
        rawrow0[j, :] = zv
        rawrow1[j, :] = zv
        return ()
    lax.fori_loop(0, _CH, zrow2, ())

    pltpu.sync_copy(zbuf, ws_sh.at[pl.ds(sid * _RPT, _RPT)])

    pltpu.sync_copy(wl_hbm, wl_v)
    pltpu.sync_copy(w2_hbm, w2_v)
    pltpu.sync_copy(b2_hbm, b2_v)
    wlk = [wl_v[pl.ds(16 * k, 16)] for k in range(8)]
    w2k = [w2_v[pl.ds(16 * k, 16)] for k in range(8)]
    b2s = b2_v[...][0]
    lanes = jnp.arange(16, dtype=jnp.int32)
    zeros_i = jnp.zeros((16,), jnp.int32)

    plsc.subcore_barrier()

    def prime(b, chunk):
        pltpu.sync_copy(edata_hbm.at[chunk], ebuf[b])
        pltpu.async_copy(ps_hbm.at[ebuf[b].at[0]], psb[b], sgp[b])
        pltpu.async_copy(pd_hbm.at[ebuf[b].at[1]], pdb[b], sgd[b])

    def wait_gathers(b):
        pltpu.make_async_copy(ps_hbm.at[ebuf[b].at[0]], psb[b], sgp[b]).wait()
        pltpu.make_async_copy(pd_hbm.at[ebuf[b].at[1]], pdb[b], sgd[b]).wait()

    def compute(b, chunk, guard):
        @pl.when(guard)
        def _():
            pltpu.make_async_copy(
                rawlinb[b], raw_hbm.at[pl.ds(0, _CH)], sst[b]).wait()
            pltpu.make_async_copy(
                rawrowb[b], ws_sh.at[didxb[b]], ssc[b]).wait()

        for i in range(8):
            ewf[pl.ds(16 * i, 16)] = plsc.bitcast(
                ebuf[b][2, pl.ds(16 * i, 16)], jnp.float32)
            didxb[b][pl.ds(16 * i, 16)] = ebuf[b][1, pl.ds(16 * i, 16)]

        def edge_body(e):
            ew_e = ewf[pl.ds(e, 16)][0]
            acc = jnp.zeros((16,), jnp.float32)
            for kk in range(8):
                a = psb[b][e, pl.ds(16 * kk, 16)]
                bb = pdb[b][e, pl.ds(16 * kk, 16)]
                t = jnp.maximum(a + bb + ew_e * wlk[kk], 0.0)
                acc = acc + t * w2k[kk]
            xacc[e, pl.ds(0, 16)] = acc
        plsc.parallel_loop(0, _CH, unroll=2)(edge_body)

        for i in range(8):
            rows = lanes + 16 * i
            s = jnp.zeros((16,), jnp.float32)
            for j in range(16):
                s = s + plsc.load_gather(xacc, [rows, zeros_i + j])
            x = s + b2s
            rv = 1.0 / (1.0 + jnp.exp(-x))
            rawlinb[b][pl.ds(16 * i, 16)] = rv
            plsc.store_scatter(rawrowb[b], [rows, zeros_i], rv)

        pltpu.async_copy(rawlinb[b], raw_hbm.at[pl.ds(chunk * _CH, _CH)],
                         sst[b])
        pltpu.async_copy(rawrowb[b], ws_sh.at[didxb[b]], ssc[b], add=True)

    prime(0, wid)
    prime(1, wid + _NW)

    def outer(k2, _):
        for b in range(2):
            k = 2 * k2 + b
            chunk = wid + _NW * k
            wait_gathers(b)
            compute(b, chunk, k2 >= 1)

            @pl.when(k + 2 < _CBASE)
            def _():
                prime(b, chunk + 2 * _NW)
        return ()
    lax.fori_loop(0, _CBASE // 2, outer, ())

    for b in range(2):
        pltpu.make_async_copy(
            rawlinb[b], raw_hbm.at[pl.ds(0, _CH)], sst[b]).wait()
        pltpu.make_async_copy(
            rawrowb[b], ws_sh.at[didxb[b]], ssc[b]).wait()

    @pl.when(wid < _CREM)
    def _():
        chunk = _NW * _CBASE + wid
        prime(0, chunk)
        wait_gathers(0)
        compute(0, chunk, wid < 0)
        pltpu.make_async_copy(
            rawlinb[0], raw_hbm.at[pl.ds(0, _CH)], sst[0]).wait()
        pltpu.make_async_copy(
            rawrowb[0], ws_sh.at[didxb[0]], ssc[0]).wait()

    plsc.subcore_barrier()

    @pl.when(sid == 0)
    def _():
        pltpu.sync_copy(ws_sh, ws_out.at[cid])


def _sc_a(edata, ps, pd, wl, w2, b2):
    mesh = plsc.VectorSubcoreMesh(core_axis_name="c", subcore_axis_name="s")
    fn = pl.kernel(
        _sc_a_body,
        out_type=[
            jax.ShapeDtypeStruct((_E,), jnp.float32),
            jax.ShapeDtypeStruct((_NC, _N, 16), jnp.float32),
        ],
        mesh=mesh,
        scratch_types=[
            pltpu.VMEM((4, _CH), jnp.int32),
            pltpu.VMEM((4, _CH), jnp.int32),
            pltpu.VMEM((_CH, _FD), jnp.float32),
            pltpu.VMEM((_CH, _FD), jnp.float32),
            pltpu.VMEM((_CH, _FD), jnp.float32),
            pltpu.VMEM((_CH, _FD), jnp.float32),
            pltpu.VMEM((_CH,), jnp.int32),
            pltpu.VMEM((_CH,), jnp.int32),
            pltpu.VMEM((_CH + 16,), jnp.float32),
            pltpu.VMEM((_CH, 17), jnp.float32),
            pltpu.VMEM((_CH,), jnp.float32),
            pltpu.VMEM((_CH,), jnp.float32),
            pltpu.VMEM((_CH, 16), jnp.float32),
            pltpu.VMEM((_CH, 16), jnp.float32),
            pltpu.VMEM((_FD,), jnp.float32),
            pltpu.VMEM((_FD,), jnp.float32),
            pltpu.VMEM((16,), jnp.float32),
            pltpu.VMEM((_RPT, 16), jnp.float32),
            pltpu.VMEM_SHARED((_N, 16), jnp.float32),
        ] + [pltpu.SemaphoreType.DMA] * 8,
        compiler_params=pltpu.CompilerParams(needs_layout_passes=False, use_tc_tiling_on_sc=False),
    )
    return fn(edata, ps, pd, wl, w2, b2)


def _sc_b_body(sidx_hbm, didx_hbm, g_hbm, s_hbm, v_hbm,
               acc_out,
               sidx_v, didx_v, g_v, sbuf, dbuf, srcrow_v, v_v, zbuf, acc_sh,
               sem1, sem2):
    cid = lax.axis_index("c")
    sid = lax.axis_index("s")
    wid = sid * _NC + cid

    zv = jnp.zeros((16,), jnp.float32)

    def zrow(j, _):
        for kk in range(4):
            zbuf[j, pl.ds(16 * kk, 16)] = zv
        return ()
    lax.fori_loop(0, _RPT, zrow, ())
    pltpu.sync_copy(zbuf, acc_sh.at[pl.ds(sid * _RPT, _RPT)])
    plsc.subcore_barrier()

    pltpu.sync_copy(v_hbm, v_v)
    vlo = v_v[pl.ds(0, 16)]
    vhi = v_v[pl.ds(16, 16)]
    lanes = jnp.arange(16, dtype=jnp.int32)
    degv = jnp.where(lanes == _Q - 16, 1.0, 0.0).astype(jnp.float32)

    nch = jnp.where(wid < _CREM, _CBASE + 1, _CBASE)

    def chunk_body(k, _):
        chunk = wid + _NW * k
        ebase = chunk * _CH
        pltpu.sync_copy(sidx_hbm.at[pl.ds(ebase, _CH)], sidx_v)
        pltpu.sync_copy(didx_hbm.at[pl.ds(ebase, _CH)], didx_v)
        pltpu.sync_copy(g_hbm.at[pl.ds(ebase, _CH)], g_v.at[pl.ds(0, _CH)])
        cp1 = pltpu.async_copy(s_hbm.at[sidx_v], sbuf, sem1)
        cp2 = pltpu.async_copy(s_hbm.at[didx_v], dbuf, sem2)
        cp1.wait()
        cp2.wait()

        def edge_body(e, _):
            fs_lo = sbuf[e, pl.ds(0, 16)]
            fs_hi = sbuf[e, pl.ds(16, 16)]
            c = g_v[pl.ds(e, 16)][0] * fs_hi[15]
            fd_lo = dbuf[e, pl.ds(0, 16)]
            fd_hi = dbuf[e, pl.ds(16, 16)]
            srcrow_v[e, pl.ds(0, 16)] = (fd_lo - fs_lo) * vlo * c
            srcrow_v[e, pl.ds(16, 16)] = (fd_hi - fs_hi) * vhi * c
            srcrow_v[e, pl.ds(32, 16)] = fs_lo
            srcrow_v[e, pl.ds(48, 16)] = fs_hi + degv
            return ()
        lax.fori_loop(0, _CH, edge_body, ())

        pltpu.sync_copy(srcrow_v, acc_sh.at[didx_v], add=True)
        return ()

    lax.fori_loop(0, nch, chunk_body, ())

    plsc.subcore_barrier()

    @pl.when(sid == 0)
    def _():
        pltpu.sync_copy(acc_sh, acc_out.at[cid])


def _sc_b(sidx, didx, g, s_tab, vpad):
    mesh = plsc.VectorSubcoreMesh(core_axis_name="c", subcore_axis_name="s")
    fn = pl.kernel(
        _sc_b_body,
        out_type=jax.ShapeDtypeStruct((_NC, _N, 64), jnp.float32),
        mesh=mesh,
        scratch_types=[
            pltpu.VMEM((_CH,), jnp.int32),
            pltpu.VMEM((_CH,), jnp.int32),
            pltpu.VMEM((_CH + 16,), jnp.float32),
            pltpu.VMEM((_CH, 32), jnp.float32),
            pltpu.VMEM((_CH, 32), jnp.float32),
            pltpu.VMEM((_CH, 64), jnp.float32),
            pltpu.VMEM((32,), jnp.float32),
            pltpu.VMEM((_RPT, 64), jnp.float32),
            pltpu.VMEM_SHARED((_N, 64), jnp.float32),
            pltpu.SemaphoreType.DMA,
            pltpu.SemaphoreType.DMA,
        ],
        compiler_params=pltpu.CompilerParams(needs_layout_passes=False, use_tc_tiling_on_sc=False),
    )
    return fn(sidx, didx, g, s_tab, vpad)


@jax.jit
def kernel(f, node_features, edge_index, edge_weight, w_fw1, b_fw1, w_fw2,
           b_fw2, w_se1, b_se1, w_se2, b_se2, w_cn1, b_cn1, w_cn2, b_cn2,
           w_cn3, b_cn3):
    sidx = edge_index[0]
    didx = edge_index[1]

    wsT = w_fw1[:, :_FD].T
    wdT = w_fw1[:, _FD:2 * _FD].T
    wl = w_fw1[:, 2 * _FD]
    w2 = w_fw2[0]
    b2 = jnp.broadcast_to(b_fw2, (16,))

    ps, pd = _tc_proj(node_features, wsT, wdT, b_fw1.reshape(1, _FD))

    edata = jnp.stack(
        [sidx.reshape(_NCHUNK, _CH),
         didx.reshape(_NCHUNK, _CH),
         jax.lax.bitcast_convert_type(edge_weight, jnp.int32).reshape(
             _NCHUNK, _CH),
         jnp.zeros((_NCHUNK, _CH), jnp.int32)],
        axis=1,
    )
    raw, ws_part = _sc_a(edata, ps, pd, wl, w2, b2)

    raw2d = raw.reshape(_E // _FD, _FD)
    ew2d = edge_weight.reshape(_E // _FD, _FD)
    s_tab, g2d = _tc_mid(f, ws_part, raw2d, ew2d)
    g = g2d.reshape(_E)

    vpad = jnp.concatenate(
        [jnp.linspace(0.0, _MAX_V, _Q, dtype=jnp.float32),
         jnp.zeros((12,), jnp.float32)]
    )

    acc = _sc_b(sidx, didx, g, s_tab, vpad)

    f_new = _tc_final(
        acc, f,
        w_se1.T, b_se1.reshape(1, _H), w_se2.T, b_se2.reshape(1, _H),
        w_cn1.T, b_cn1.reshape(1, _H), w_cn2.T, b_cn2.reshape(1, _H),
        w_cn3.T, b_cn3.reshape(1, _Q),
    )
    return f_new


# speedup vs baseline: 13.9396x; 1.5561x over previous
import functools

import jax
import jax.numpy as jnp
from jax import lax
from jax.experimental import pallas as pl
from jax.experimental.pallas import tpu as pltpu
from jax.experimental.pallas import tpu_sc as plsc

_N = 10000
_E = 320000
_Q = 20
_FD = 128
_H = 64
_MAX_V = 20.0
_DT = 0.1

_NC = 2
_NS = 16
_NW = _NC * _NS
_CH = 128
_NCHUNK = _E // _CH
_CBASE = _NCHUNK // _NW
_CREM = _NCHUNK - _CBASE * _NW
_RPT = _N // _NS


def _tc_proj_body(x_ref, wsT_ref, wdT_ref, b1_ref, ps_ref, pd_ref):
    x = x_ref[...]
    ps_ref[...] = (
        jnp.dot(x, wsT_ref[...], precision=jax.lax.Precision.HIGHEST)
        + b1_ref[...]
    )
    pd_ref[...] = jnp.dot(x, wdT_ref[...], precision=jax.lax.Precision.HIGHEST)


def _tc_proj(nf, wsT, wdT, b1):
    blk = 1000
    grid = _N // blk
    return pl.pallas_call(
        _tc_proj_body,
        grid=(grid,),
        in_specs=[
            pl.BlockSpec((blk, _FD), lambda i: (i, 0)),
            pl.BlockSpec((_FD, _FD), lambda i: (0, 0)),
            pl.BlockSpec((_FD, _FD), lambda i: (0, 0)),
            pl.BlockSpec((1, _FD), lambda i: (0, 0)),
        ],
        out_specs=[
            pl.BlockSpec((blk, _FD), lambda i: (i, 0)),
            pl.BlockSpec((blk, _FD), lambda i: (i, 0)),
        ],
        out_shape=[
            jax.ShapeDtypeStruct((_N, _FD), jnp.float32),
            jax.ShapeDtypeStruct((_N, _FD), jnp.float32),
        ],
    )(nf, wsT, wdT, b1)


def _tc_mid_body(f_ref, ws_ref, raw_ref, ew_ref, s_ref, g_ref):
    ws = ws_ref[0, :, 0:1] + ws_ref[1, :, 0:1]
    inv = 1.0 / jnp.maximum(ws, 1e-8)
    f = f_ref[...]
    s_ref[...] = jnp.concatenate(
        [f, jnp.zeros((f.shape[0], 32 - _Q - 1), jnp.float32), inv], axis=-1
    )
    g_ref[...] = raw_ref[...] / jnp.maximum(ew_ref[...], 1e-8)


def _tc_mid(f, ws_part, raw2d, ew2d):
    return pl.pallas_call(
        _tc_mid_body,
        out_shape=[
            jax.ShapeDtypeStruct((_N, 32), jnp.float32),
            jax.ShapeDtypeStruct(raw2d.shape, jnp.float32),
        ],
    )(f, ws_part, raw2d, ew2d)


def _tc_final_body(acc_ref, f_ref, wse1_ref, bse1_ref, wse2_ref, bse2_ref,
                   wcn1_ref, bcn1_ref, wcn2_ref, bcn2_ref, wcn3_ref, bcn3_ref,
                   out_ref):
    a = acc_ref[0] + acc_ref[1]
    transport = a[:, 0:_Q]
    neigh = a[:, 32:32 + _Q]
    deg = a[:, 52:53]
    f = f_ref[...]
    hp = jax.lax.Precision.HIGHEST
    nf_avg = neigh / jnp.maximum(deg, 1.0)
    sp = jnp.maximum(jnp.dot(nf_avg, wse1_ref[...], precision=hp)
                     + bse1_ref[...], 0.0)
    sp = jnp.dot(sp, wse2_ref[...], precision=hp) + bse2_ref[...]
    ci = jnp.concatenate([f, sp], axis=-1)
    o = jnp.maximum(jnp.dot(ci, wcn1_ref[...], precision=hp)
                    + bcn1_ref[...], 0.0)
    o = jnp.maximum(jnp.dot(o, wcn2_ref[...], precision=hp)
                    + bcn2_ref[...], 0.0)
    omega = jnp.dot(o, wcn3_ref[...], precision=hp) + bcn3_ref[...]
    out_ref[...] = f + _DT * (omega - transport)


def _tc_final(acc, f, wse1T, bse1, wse2T, bse2, wcn1T, bcn1, wcn2T, bcn2,
              wcn3T, bcn3):
    blk = 1000
    grid = _N // blk
    wspec = lambda shape: pl.BlockSpec(shape, lambda i: tuple(0 for _ in shape))
    return pl.pallas_call(
        _tc_final_body,
        grid=(grid,),
        in_specs=[
            pl.BlockSpec((2, blk, 64), lambda i: (0, i, 0)),
            pl.BlockSpec((blk, _Q), lambda i: (i, 0)),
            wspec(wse1T.shape), wspec(bse1.shape),
            wspec(wse2T.shape), wspec(bse2.shape),
            wspec(wcn1T.shape), wspec(bcn1.shape),
            wspec(wcn2T.shape), wspec(bcn2.shape),
            wspec(wcn3T.shape), wspec(bcn3.shape),
        ],
        out_specs=pl.BlockSpec((blk, _Q), lambda i: (i, 0)),
        out_shape=jax.ShapeDtypeStruct((_N, _Q), jnp.float32),
    )(acc, f, wse1T, bse1, wse2T, bse2, wcn1T, bcn1, wcn2T, bcn2, wcn3T, bcn3)


def _sc_a_body(edata_hbm, ps_hbm, pd_hbm, wl_hbm, w2_hbm, b2_hbm,
               raw_hbm, ws_out,
               ebuf0, ebuf1, ps0, ps1, pd0, pd1, didx0, didx1, ewf,
               xacc, rawlin0, rawlin1, rawrow0, rawrow1,
               wl_v, w2_v, b2_v, zbuf, ws_sh,
               sgp0, sgp1, sgd0, sgd1, sst0, sst1, ssc0, ssc1):
    cid = lax.axis_index("c")
    sid = lax.axis_index("s")
    wid = sid * _NC + cid

    ebuf = [ebuf0, ebuf1]
    psb = [ps0, ps1]
    pdb = [pd0, pd1]
    didxb = [didx0, didx1]
    rawlinb = [rawlin0, rawlin1]
    rawrowb = [rawrow0, rawrow1]
    sgp = [sgp0, sgp1]
    sgd = [sgd0, sgd1]
    sst = [sst0, sst1]
    ssc = [ssc0, ssc1]

    zv = jnp.zeros((16,), jnp.float32)

    def zrow(j, _):
        zbuf[j, :] = zv
        return ()
    lax.fori_loop(0, _RPT, zrow, ())

    def zrow2(j, _):
        rawrow0[j, :] = zv
        rawrow1[j, :] = zv
        return ()
    lax.fori_loop(0, _CH, zrow2, ())

    pltpu.sync_copy(zbuf, ws_sh.at[pl.ds(sid * _RPT, _RPT)])

    pltpu.sync_copy(wl_hbm, wl_v)
    pltpu.sync_copy(w2_hbm, w2_v)
    pltpu.sync_copy(b2_hbm, b2_v)
    wlk = [wl_v[pl.ds(16 * k, 16)] for k in range(8)]
    w2k = [w2_v[pl.ds(16 * k, 16)] for k in range(8)]
    b2s = b2_v[...][0]
    lanes = jnp.arange(16, dtype=jnp.int32)
    zeros_i = jnp.zeros((16,), jnp.int32)

    plsc.subcore_barrier()

    def prime(b, chunk):
        pltpu.sync_copy(edata_hbm.at[chunk], ebuf[b])
        pltpu.async_copy(ps_hbm.at[ebuf[b].at[0]], psb[b], sgp[b])
        pltpu.async_copy(pd_hbm.at[ebuf[b].at[1]], pdb[b], sgd[b])

    def wait_gathers(b):
        pltpu.make_async_copy(ps_hbm.at[ebuf[b].at[0]], psb[b], sgp[b]).wait()
        pltpu.make_async_copy(pd_hbm.at[ebuf[b].at[1]], pdb[b], sgd[b]).wait()

    def compute(b, chunk, guard):
        @pl.when(guard)
        def _():
            pltpu.make_async_copy(
                rawlinb[b], raw_hbm.at[pl.ds(0, _CH)], sst[b]).wait()
            pltpu.make_async_copy(
                rawrowb[b], ws_sh.at[didxb[b]], ssc[b]).wait()

        for i in range(8):
            ewf[pl.ds(16 * i, 16)] = plsc.bitcast(
                ebuf[b][2, pl.ds(16 * i, 16)], jnp.float32)
            didxb[b][pl.ds(16 * i, 16)] = ebuf[b][1, pl.ds(16 * i, 16)]

        def edge_body(e):
            ew_e = ewf[pl.ds(e, 16)][0]
            acc = jnp.zeros((16,), jnp.float32)
            for kk in range(8):
                a = psb[b][e, pl.ds(16 * kk, 16)]
                bb = pdb[b][e, pl.ds(16 * kk, 16)]
                t = jnp.maximum(a + bb + ew_e * wlk[kk], 0.0)
                acc = acc + t * w2k[kk]
            xacc[e, pl.ds(0, 16)] = acc
        plsc.parallel_loop(0, _CH, unroll=2)(edge_body)

        for i in range(8):
            rows = lanes + 16 * i
            s = jnp.zeros((16,), jnp.float32)
            for j in range(16):
                s = s + plsc.load_gather(xacc, [rows, zeros_i + j])
            x = s + b2s
            rv = 1.0 / (1.0 + jnp.exp(-x))
            rawlinb[b][pl.ds(16 * i, 16)] = rv
            plsc.store_scatter(rawrowb[b], [rows, zeros_i], rv)

        pltpu.async_copy(rawlinb[b], raw_hbm.at[pl.ds(chunk * _CH, _CH)],
                         sst[b])
        pltpu.async_copy(rawrowb[b], ws_sh.at[didxb[b]], ssc[b], add=True)

    prime(0, wid)
    prime(1, wid + _NW)

    def outer(k2, _):
        for b in range(2):
            k = 2 * k2 + b
            chunk = wid + _NW * k
            wait_gathers(b)
            compute(b, chunk, k2 >= 1)

            @pl.when(k + 2 < _CBASE)
            def _():
                prime(b, chunk + 2 * _NW)
        return ()
    lax.fori_loop(0, _CBASE // 2, outer, ())

    for b in range(2):
        pltpu.make_async_copy(
            rawlinb[b], raw_hbm.at[pl.ds(0, _CH)], sst[b]).wait()
        pltpu.make_async_copy(
            rawrowb[b], ws_sh.at[didxb[b]], ssc[b]).wait()

    @pl.when(wid < _CREM)
    def _():
        chunk = _NW * _CBASE + wid
        prime(0, chunk)
        wait_gathers(0)
        compute(0, chunk, wid < 0)
        pltpu.make_async_copy(
            rawlinb[0], raw_hbm.at[pl.ds(0, _CH)], sst[0]).wait()
        pltpu.make_async_copy(
            rawrowb[0], ws_sh.at[didxb[0]], ssc[0]).wait()

    plsc.subcore_barrier()

    @pl.when(sid == 0)
    def _():
        pltpu.sync_copy(ws_sh, ws_out.at[cid])


def _sc_a(edata, ps, pd, wl, w2, b2):
    mesh = plsc.VectorSubcoreMesh(core_axis_name="c", subcore_axis_name="s")
    fn = pl.kernel(
        _sc_a_body,
        out_type=[
            jax.ShapeDtypeStruct((_E,), jnp.float32),
            jax.ShapeDtypeStruct((_NC, _N, 16), jnp.float32),
        ],
        mesh=mesh,
        scratch_types=[
            pltpu.VMEM((4, _CH), jnp.int32),
            pltpu.VMEM((4, _CH), jnp.int32),
            pltpu.VMEM((_CH, _FD), jnp.float32),
            pltpu.VMEM((_CH, _FD), jnp.float32),
            pltpu.VMEM((_CH, _FD), jnp.float32),
            pltpu.VMEM((_CH, _FD), jnp.float32),
            pltpu.VMEM((_CH,), jnp.int32),
            pltpu.VMEM((_CH,), jnp.int32),
            pltpu.VMEM((_CH + 16,), jnp.float32),
            pltpu.VMEM((_CH, 17), jnp.float32),
            pltpu.VMEM((_CH,), jnp.float32),
            pltpu.VMEM((_CH,), jnp.float32),
            pltpu.VMEM((_CH, 16), jnp.float32),
            pltpu.VMEM((_CH, 16), jnp.float32),
            pltpu.VMEM((_FD,), jnp.float32),
            pltpu.VMEM((_FD,), jnp.float32),
            pltpu.VMEM((16,), jnp.float32),
            pltpu.VMEM((_RPT, 16), jnp.float32),
            pltpu.VMEM_SHARED((_N, 16), jnp.float32),
        ] + [pltpu.SemaphoreType.DMA] * 8,
        compiler_params=pltpu.CompilerParams(needs_layout_passes=False, use_tc_tiling_on_sc=False),
    )
    return fn(edata, ps, pd, wl, w2, b2)


def _sc_b_body(edata_hbm, g_hbm, s_hbm, v_hbm,
               acc_out,
               ebuf0, ebuf1, sb0, sb1, db0, db1, didx0, didx1, g0, g1,
               srcrow0, srcrow1, v_v, zbuf, acc_sh,
               sgp0, sgp1, sgd0, sgd1, ssc0, ssc1):
    cid = lax.axis_index("c")
    sid = lax.axis_index("s")
    wid = sid * _NC + cid

    ebuf = [ebuf0, ebuf1]
    sbb = [sb0, sb1]
    dbb = [db0, db1]
    didxb = [didx0, didx1]
    gb = [g0, g1]
    srcrowb = [srcrow0, srcrow1]
    sgp = [sgp0, sgp1]
    sgd = [sgd0, sgd1]
    ssc = [ssc0, ssc1]

    zv = jnp.zeros((16,), jnp.float32)

    def zrow(j, _):
        for kk in range(4):
            zbuf[j, pl.ds(16 * kk, 16)] = zv
        return ()
    lax.fori_loop(0, _RPT, zrow, ())
    pltpu.sync_copy(zbuf, acc_sh.at[pl.ds(sid * _RPT, _RPT)])

    pltpu.sync_copy(v_hbm, v_v)
    vlo = v_v[pl.ds(0, 16)]
    vhi = v_v[pl.ds(16, 16)]
    lanes = jnp.arange(16, dtype=jnp.int32)
    degv = jnp.where(lanes == _Q - 16, 1.0, 0.0).astype(jnp.float32)

    plsc.subcore_barrier()

    def prime(b, chunk):
        pltpu.sync_copy(edata_hbm.at[chunk], ebuf[b])
        pltpu.sync_copy(g_hbm.at[chunk], gb[b].at[pl.ds(0, _CH)])
        pltpu.async_copy(s_hbm.at[ebuf[b].at[0]], sbb[b], sgp[b])
        pltpu.async_copy(s_hbm.at[ebuf[b].at[1]], dbb[b], sgd[b])

    def wait_gathers(b):
        pltpu.make_async_copy(s_hbm.at[ebuf[b].at[0]], sbb[b], sgp[b]).wait()
        pltpu.make_async_copy(s_hbm.at[ebuf[b].at[1]], dbb[b], sgd[b]).wait()

    def compute(b, guard):
        @pl.when(guard)
        def _():
            pltpu.make_async_copy(
                srcrowb[b], acc_sh.at[didxb[b]], ssc[b]).wait()

        for i in range(8):
            didxb[b][pl.ds(16 * i, 16)] = ebuf[b][1, pl.ds(16 * i, 16)]

        def edge_body(e):
            fs_lo = sbb[b][e, pl.ds(0, 16)]
            fs_hi = sbb[b][e, pl.ds(16, 16)]
            c = gb[b][pl.ds(e, 16)][0] * fs_hi[15]
            fd_lo = dbb[b][e, pl.ds(0, 16)]
            fd_hi = dbb[b][e, pl.ds(16, 16)]
            srcrowb[b][e, pl.ds(0, 16)] = (fd_lo - fs_lo) * vlo * c
            srcrowb[b][e, pl.ds(16, 16)] = (fd_hi - fs_hi) * vhi * c
            srcrowb[b][e, pl.ds(32, 16)] = fs_lo
            srcrowb[b][e, pl.ds(48, 16)] = fs_hi + degv
        plsc.parallel_loop(0, _CH, unroll=2)(edge_body)

        pltpu.async_copy(srcrowb[b], acc_sh.at[didxb[b]], ssc[b], add=True)

    prime(0, wid)
    prime(1, wid + _NW)

    def outer(k2, _):
        for b in range(2):
            k = 2 * k2 + b
            chunk = wid + _NW * k
            wait_gathers(b)
            compute(b, k2 >= 1)

            @pl.when(k + 2 < _CBASE)
            def _():
                prime(b, chunk + 2 * _NW)
        return ()
    lax.fori_loop(0, _CBASE // 2, outer, ())

    for b in range(2):
        pltpu.make_async_copy(srcrowb[b], acc_sh.at[didxb[b]], ssc[b]).wait()

    @pl.when(wid < _CREM)
    def _():
        chunk = _NW * _CBASE + wid
        prime(0, chunk)
        wait_gathers(0)
        compute(0, wid < 0)
        pltpu.make_async_copy(srcrowb[0], acc_sh.at[didxb[0]], ssc[0]).wait()

    plsc.subcore_barrier()

    @pl.when(sid == 0)
    def _():
        pltpu.sync_copy(acc_sh, acc_out.at[cid])


def _sc_b(edata, g2d, s_tab, vpad):
    mesh = plsc.VectorSubcoreMesh(core_axis_name="c", subcore_axis_name="s")
    fn = pl.kernel(
        _sc_b_body,
        out_type=jax.ShapeDtypeStruct((_NC, _N, 64), jnp.float32),
        mesh=mesh,
        scratch_types=[
            pltpu.VMEM((4, _CH), jnp.int32),
            pltpu.VMEM((4, _CH), jnp.int32),
            pltpu.VMEM((_CH, 32), jnp.float32),
            pltpu.VMEM((_CH, 32), jnp.float32),
            pltpu.VMEM((_CH, 32), jnp.float32),
            pltpu.VMEM((_CH, 32), jnp.float32),
            pltpu.VMEM((_CH,), jnp.int32),
            pltpu.VMEM((_CH,), jnp.int32),
            pltpu.VMEM((_CH + 16,), jnp.float32),
            pltpu.VMEM((_CH + 16,), jnp.float32),
            pltpu.VMEM((_CH, 64), jnp.float32),
            pltpu.VMEM((_CH, 64), jnp.float32),
            pltpu.VMEM((32,), jnp.float32),
            pltpu.VMEM((_RPT, 64), jnp.float32),
            pltpu.VMEM_SHARED((_N, 64), jnp.float32),
        ] + [pltpu.SemaphoreType.DMA] * 6,
        compiler_params=pltpu.CompilerParams(needs_layout_passes=False, use_tc_tiling_on_sc=False),
    )
    return fn(edata, g2d, s_tab, vpad)


@jax.jit
def kernel(f, node_features, edge_index, edge_weight, w_fw1, b_fw1, w_fw2,
           b_fw2, w_se1, b_se1, w_se2, b_se2, w_cn1, b_cn1, w_cn2, b_cn2,
           w_cn3, b_cn3):
    sidx = edge_index[0]
    didx = edge_index[1]

    wsT = w_fw1[:, :_FD].T
    wdT = w_fw1[:, _FD:2 * _FD].T
    wl = w_fw1[:, 2 * _FD]
    w2 = w_fw2[0]
    b2 = jnp.broadcast_to(b_fw2, (16,))

    ps, pd = _tc_proj(node_features, wsT, wdT, b_fw1.reshape(1, _FD))

    edata = jnp.stack(
        [sidx.reshape(_NCHUNK, _CH),
         didx.reshape(_NCHUNK, _CH),
         jax.lax.bitcast_convert_type(edge_weight, jnp.int32).reshape(
             _NCHUNK, _CH),
         jnp.zeros((_NCHUNK, _CH), jnp.int32)],
        axis=1,
    )
    raw, ws_part = _sc_a(edata, ps, pd, wl, w2, b2)

    raw2d = raw.reshape(_E // _FD, _FD)
    ew2d = edge_weight.reshape(_E // _FD, _FD)
    s_tab, g2d = _tc_mid(f, ws_part, raw2d, ew2d)

    vpad = jnp.concatenate(
        [jnp.linspace(0.0, _MAX_V, _Q, dtype=jnp.float32),
         jnp.zeros((12,), jnp.float32)]
    )

    acc = _sc_b(edata, g2d, s_tab, vpad)

    f_new = _tc_final(
        acc, f,
        w_se1.T, b_se1.reshape(1, _H), w_se2.T, b_se2.reshape(1, _H),
        w_cn1.T, b_cn1.reshape(1, _H), w_cn2.T, b_cn2.reshape(1, _H),
        w_cn3.T, b_cn3.reshape(1, _Q),
    )
    return f_new
